# baseline (speedup 1.0000x reference)
import functools

import jax
import jax.numpy as jnp
import numpy as np
from jax import lax
from jax.experimental import pallas as pl
from jax.experimental.pallas import tpu as pltpu
from jax.experimental.pallas import tpu_sc as plsc

N = 10000
E = 320000
R = 4
F = 128
OUT = 64
EPR = E // R

NC = 2
NS = 16
EPT = EPR // NS
NPAD = 10240
DUMMY = N
OPT = NPAD // NS

CHD = 128
NCHD = EPT // CHD + 1
NPTD = NCHD * CHD

SLT = N // NS
CH = 64
NCH = 88
NPT = NCH * CH

BLK = 1000
GRID = N // BLK

_MP = np.zeros(F, np.int32)
for _g in range(F // 32):
    for _k in range(16):
        _MP[32 * _g + _k] = 32 * _g + 2 * _k
        _MP[32 * _g + 16 + _k] = 32 * _g + 2 * _k + 1
_COLQ = np.argsort(_MP)

_mesh = functools.partial(
    plsc.VectorSubcoreMesh, core_axis_name="c", subcore_axis_name="s"
)


def _for_each_owned_relation(c, fn):
    for k in range(2):
        for core_id in range(NC):
            r = core_id * 2 + k

            @pl.when(c == core_id)
            def _(r=r):
                fn(r)


def _deg_body(dstp_hbm, out_hbm, didx, ones, zb, acc1, sem):
    del sem
    c = lax.axis_index("c")
    s = lax.axis_index("s")
    one16 = jnp.full((16,), 1.0, jnp.float32)
    zero16 = jnp.zeros((16,), jnp.float32)
    for i in range(CHD):
        ones[i, :] = one16
    for i in range(16):
        zb[i, :] = zero16

    def run(r):
        def zbody(i, _):
            pltpu.sync_copy(zb, acc1.at[pl.ds(s * OPT + i * 16, 16)])
            return 0

        lax.fori_loop(0, OPT // 16, zbody, 0)
        pltpu.sync_copy(dstp_hbm.at[pl.ds((r * NS + s) * NCHD, NCHD)], didx)
        plsc.subcore_barrier()

        def body(j, _):
            pltpu.sync_copy(ones, acc1.at[didx.at[j]], add=True)
            return 0

        lax.fori_loop(0, NCHD, body, 0)
        plsc.subcore_barrier()
        pltpu.sync_copy(
            acc1.at[pl.ds(s * OPT, OPT)],
            out_hbm.at[pl.ds(r * NPAD + s * OPT, OPT)],
        )
        plsc.subcore_barrier()

    _for_each_owned_relation(c, run)


def _deg_kernel(dst_pad):
    return pl.kernel(
        _deg_body,
        out_type=jax.ShapeDtypeStruct((R * NPAD, 16), jnp.float32),
        mesh=_mesh(),
        scratch_types=[
            pltpu.VMEM((NCHD, CHD), jnp.int32),
            pltpu.VMEM((CHD, 16), jnp.float32),
            pltpu.VMEM((16, 16), jnp.float32),
            pltpu.VMEM_SHARED((NPAD, 16), jnp.float32),
            pltpu.SemaphoreType.DMA,
        ],
    )(dst_pad)


def _gs_body(
    tbl_hbm, srcp_hbm, dstp_hbm, out_hbm,
    sidx, didx, g0, g1, s0, s1, acc, sga, sgb, ssa, ssb,
):
    c = lax.axis_index("c")
    s = lax.axis_index("s")
    zero16 = jnp.zeros((16,), jnp.float32)

    def conv(gr, sr):
        def crow(i, _):
            for g in range(F // 32):
                w = gr[i, pl.ds(16 * g, 16)]
                lo = plsc.bitcast(lax.shift_left(w, 16), jnp.float32)
                hi = plsc.bitcast(
                    lax.shift_left(lax.shift_right_logical(w, 16), 16),
                    jnp.float32,
                )
                sr[i, pl.ds(32 * g, 16)] = lo
                sr[i, pl.ds(32 * g + 16, 16)] = hi
            return 0

        lax.fori_loop(0, CH, crow, 0)

    def run(r):
        def zfill(i, _):
            for j2 in range(F // 16):
                s0[i, pl.ds(j2 * 16, 16)] = zero16
            return 0

        lax.fori_loop(0, CH, zfill, 0)

        def zbody(i, _):
            pltpu.sync_copy(s0, acc.at[pl.ds(s * OPT + i * CH, CH)])
            return 0

        lax.fori_loop(0, OPT // CH, zbody, 0)
        pltpu.sync_copy(srcp_hbm.at[pl.ds((r * NS + s) * NCH, NCH)], sidx)
        pltpu.sync_copy(dstp_hbm.at[pl.ds((r * NS + s) * NCH, NCH)], didx)
        plsc.subcore_barrier()

        def body(i, _):
            j0 = 2 * i
            j1 = 2 * i + 1
            d0 = pltpu.async_copy(tbl_hbm.at[sidx.at[j0]], g0, sga)
            d1 = pltpu.async_copy(tbl_hbm.at[sidx.at[j1]], g1, sgb)
            d0.wait()
            conv(g0, s0)
            sc0 = pltpu.make_async_copy(s0, acc.at[didx.at[j0]], ssa)
            sc0.start(add=True)
            d1.wait()
            conv(g1, s1)
            sc1 = pltpu.make_async_copy(s1, acc.at[didx.at[j1]], ssb)
            sc1.start(add=True)
            sc0.wait()
            sc1.wait()
            return 0

        lax.fori_loop(0, NCH // 2, body, 0)
        plsc.subcore_barrier()
        pltpu.sync_copy(
            acc.at[pl.ds(s * OPT, OPT)],
            out_hbm.at[pl.ds(r * NPAD + s * OPT, OPT)],
        )
        plsc.subcore_barrier()

    _for_each_owned_relation(c, run)


def _gs_kernel(tbl_i32, src_pad, dst_pad):
    return pl.kernel(
        _gs_body,
        out_type=jax.ShapeDtypeStruct((R * NPAD, F), jnp.float32),
        mesh=_mesh(),
        compiler_params=pltpu.CompilerParams(
            use_tc_tiling_on_sc=False, needs_layout_passes=False
        ),
        scratch_types=[
            pltpu.VMEM((NCH, CH), jnp.int32),
            pltpu.VMEM((NCH, CH), jnp.int32),
            pltpu.VMEM((CH, F // 2), jnp.int32),
            pltpu.VMEM((CH, F // 2), jnp.int32),
            pltpu.VMEM((CH, F), jnp.float32),
            pltpu.VMEM((CH, F), jnp.float32),
            pltpu.VMEM_SHARED((NPAD, F), jnp.float32),
            pltpu.SemaphoreType.DMA,
            pltpu.SemaphoreType.DMA,
            pltpu.SemaphoreType.DMA,
            pltpu.SemaphoreType.DMA,
        ],
    )(tbl_i32, src_pad, dst_pad)


def _tc1_body(x_ref, wm_ref, bm_ref, wg_ref, degc_ref, h_ref, dinv_ref, tmp_ref):
    h = jnp.maximum(
        jnp.dot(x_ref[...], wm_ref[...], preferred_element_type=jnp.float32)
        + bm_ref[...],
        0.0,
    )
    h_ref[...] = h
    deg = degc_ref[...][:, :, 0] + 1.0
    dinv = lax.rsqrt(deg)
    dinv_ref[...] = dinv
    for r in range(R):
        tmp_ref[r] = (
            jnp.dot(h, wg_ref[r], preferred_element_type=jnp.float32)
            * dinv[:, r : r + 1]
        ).astype(jnp.bfloat16)


def _tc1(x, w_mlp, b_mlp, wg0, degc):
    return pl.pallas_call(
        _tc1_body,
        grid=(GRID,),
        in_specs=[
            pl.BlockSpec((BLK, F), lambda i: (i, 0)),
            pl.BlockSpec((F, F), lambda i: (0, 0)),
            pl.BlockSpec((1, F), lambda i: (0, 0)),
            pl.BlockSpec((R, F, F), lambda i: (0, 0, 0)),
            pl.BlockSpec((BLK, R, 16), lambda i: (i, 0, 0)),
        ],
        out_specs=[
            pl.BlockSpec((BLK, F), lambda i: (i, 0)),
            pl.BlockSpec((BLK, R), lambda i: (i, 0)),
            pl.BlockSpec((R, BLK, F), lambda i: (0, i, 0)),
        ],
        out_shape=[
            jax.ShapeDtypeStruct((N, F), jnp.float32),
            jax.ShapeDtypeStruct((N, R), jnp.float32),
            jax.ShapeDtypeStruct((R, N, F), jnp.bfloat16),
        ],
    )(x, w_mlp, b_mlp, wg0, degc)


def _post_body(acc_ref, dinv_ref, bg_ref, h_ref, sums_ref):
    i = pl.program_id(0)
    acc = acc_ref[...]
    dinv = dinv_ref[...]
    tot = jnp.zeros((BLK, F), jnp.float32)
    for r in range(R):
        tot = tot + acc[r] * dinv[:, r : r + 1] + bg_ref[r]
    h = tot * (1.0 / R)
    h_ref[...] = h
    stats = jnp.concatenate(
        [
            jnp.sum(h, axis=0, keepdims=True),
            jnp.sum(h * h, axis=0, keepdims=True),
        ],
        axis=0,
    )

    @pl.when(i == 0)
    def _():
        sums_ref[...] = stats

    @pl.when(i > 0)
    def _():
        sums_ref[...] += stats


def _post(acc, dinv, bg):
    return pl.pallas_call(
        _post_body,
        grid=(GRID,),
        in_specs=[
            pl.BlockSpec((R, BLK, F), lambda i: (0, i, 0)),
            pl.BlockSpec((BLK, R), lambda i: (i, 0)),
            pl.BlockSpec((R, 1, F), lambda i: (0, 0, 0)),
        ],
        out_specs=[
            pl.BlockSpec((BLK, F), lambda i: (i, 0)),
            pl.BlockSpec((2, F), lambda i: (0, 0)),
        ],
        out_shape=[
            jax.ShapeDtypeStruct((N, F), jnp.float32),
            jax.ShapeDtypeStruct((2, F), jnp.float32),
        ],
    )(acc, dinv, bg)


def _bn_next_body(hpre_ref, sums_ref, g_ref, b_ref, wg_ref, dinv_ref, tmp_ref):
    mu = sums_ref[0:1, :] * (1.0 / N)
    ex2 = sums_ref[1:2, :] * (1.0 / N)
    var = ex2 - mu * mu
    scale = g_ref[...] * lax.rsqrt(var + 1e-5)
    h = jnp.maximum((hpre_ref[...] - mu) * scale + b_ref[...], 0.0)
    dinv = dinv_ref[...]
    for r in range(R):
        tmp_ref[r] = (
            jnp.dot(h, wg_ref[r], preferred_element_type=jnp.float32)
            * dinv[:, r : r + 1]
        ).astype(jnp.bfloat16)


def _bn_next(hpre, sums, gamma, beta, wg1, dinv):
    return pl.pallas_call(
        _bn_next_body,
        grid=(GRID,),
        in_specs=[
            pl.BlockSpec((BLK, F), lambda i: (i, 0)),
            pl.BlockSpec((2, F), lambda i: (0, 0)),
            pl.BlockSpec((1, F), lambda i: (0, 0)),
            pl.BlockSpec((1, F), lambda i: (0, 0)),
            pl.BlockSpec((R, F, F), lambda i: (0, 0, 0)),
            pl.BlockSpec((BLK, R), lambda i: (i, 0)),
        ],
        out_specs=[pl.BlockSpec((R, BLK, F), lambda i: (0, i, 0))],
        out_shape=[jax.ShapeDtypeStruct((R, N, F), jnp.bfloat16)],
    )(hpre, sums, gamma, beta, wg1, dinv)[0]


def _final_body(hpre_ref, sums_ref, g_ref, b_ref, res_ref, wl_ref, bl_ref, out_ref):
    mu = sums_ref[0:1, :] * (1.0 / N)
    ex2 = sums_ref[1:2, :] * (1.0 / N)
    var = ex2 - mu * mu
    scale = g_ref[...] * lax.rsqrt(var + 1e-5)
    h = jnp.maximum((hpre_ref[...] - mu) * scale + b_ref[...], 0.0)
    h = h + res_ref[...]
    out_ref[...] = (
        jnp.dot(h, wl_ref[...], preferred_element_type=jnp.float32) + bl_ref[...]
    )


def _final(hpre, sums, gamma, beta, res, w_last, b_last):
    return pl.pallas_call(
        _final_body,
        grid=(GRID,),
        in_specs=[
            pl.BlockSpec((BLK, F), lambda i: (i, 0)),
            pl.BlockSpec((2, F), lambda i: (0, 0)),
            pl.BlockSpec((1, F), lambda i: (0, 0)),
            pl.BlockSpec((1, F), lambda i: (0, 0)),
            pl.BlockSpec((BLK, F), lambda i: (i, 0)),
            pl.BlockSpec((F, OUT), lambda i: (0, 0)),
            pl.BlockSpec((1, OUT), lambda i: (0, 0)),
        ],
        out_specs=[pl.BlockSpec((BLK, OUT), lambda i: (i, 0))],
        out_shape=[jax.ShapeDtypeStruct((N, OUT), jnp.float32)],
    )(hpre, sums, gamma, beta, res, w_last, b_last)[0]


def _to_i32_table(tbl_bf16):
    return lax.bitcast_convert_type(
        tbl_bf16.reshape(R * N, F // 2, 2), jnp.int32
    )


def kernel(x, edge_index, W_mlp, b_mlp, W_gcn, b_gcn, bn_gamma, bn_beta, W_last, b_last):
    ei = edge_index.astype(jnp.int32)
    src = ei[0].reshape(R, NS, EPT)
    dst = ei[1].reshape(R, NS, EPT)

    padd = ((0, 0), (0, 0), (0, NPTD - EPT))
    dst_pad_deg = jnp.pad(dst, padd, constant_values=DUMMY).reshape(
        R * NS * NCHD, CHD
    )

    selfe = jnp.broadcast_to(
        jnp.arange(N, dtype=jnp.int32).reshape(1, NS, SLT), (R, NS, SLT)
    )
    src_full = jnp.concatenate([src, selfe], axis=2)
    dst_full = jnp.concatenate([dst, selfe], axis=2)
    src_full = src_full + (jnp.arange(R, dtype=jnp.int32) * N)[:, None, None]
    pads = ((0, 0), (0, 0), (0, NPT - EPT - SLT))
    src_pad = jnp.pad(src_full, pads, constant_values=0).reshape(R * NS * NCH, CH)
    dst_pad = jnp.pad(dst_full, pads, constant_values=DUMMY).reshape(
        R * NS * NCH, CH
    )

    degc = _deg_kernel(dst_pad_deg).reshape(R, NPAD, 16)[:, :N].transpose(1, 0, 2)

    b_mlp2 = b_mlp.reshape(1, F)
    bg = b_gcn.reshape(2, R, 1, F)
    gamma = bn_gamma.reshape(2, 1, F)
    beta = bn_beta.reshape(2, 1, F)
    colq = jnp.asarray(_COLQ)
    wg_q = W_gcn[:, :, :, colq]

    h0, dinv, tbl1 = _tc1(x, W_mlp, b_mlp2, wg_q[0], degc)

    acc = _gs_kernel(_to_i32_table(tbl1), src_pad, dst_pad)
    acc = acc.reshape(R, NPAD, F)[:, :N]
    h1pre, sums1 = _post(acc, dinv, bg[0])
    tbl2 = _bn_next(h1pre, sums1, gamma[0], beta[0], wg_q[1], dinv)

    acc2 = _gs_kernel(_to_i32_table(tbl2), src_pad, dst_pad)
    acc2 = acc2.reshape(R, NPAD, F)[:, :N]
    h2pre, sums2 = _post(acc2, dinv, bg[1])
    return _final(
        h2pre, sums2, gamma[1], beta[1], h0, W_last, b_last.reshape(1, OUT)
    )


# speedup vs baseline: 2.1686x; 2.1686x over previous
import functools

import jax
import jax.numpy as jnp
import numpy as np
from jax import lax
from jax.experimental import pallas as pl
from jax.experimental.pallas import tpu as pltpu
from jax.experimental.pallas import tpu_sc as plsc

N = 10000
E = 320000
R = 4
F = 128
OUT = 64
EPR = E // R

NC = 2
NS = 16
EPT = EPR // NS
NPAD = 10240
DUMMY = N
OPT = NPAD // NS

CHD = 128
NCHD = EPT // CHD + 1
NPTD = NCHD * CHD

SLT = N // NS
CH = 128
NCH = 44
NPT = NCH * CH

BLK = 1000
GRID = N // BLK


_mesh = functools.partial(
    plsc.VectorSubcoreMesh, core_axis_name="c", subcore_axis_name="s"
)


def _for_each_owned_relation(c, fn):
    for k in range(2):
        for core_id in range(NC):
            r = core_id * 2 + k

            @pl.when(c == core_id)
            def _(r=r):
                fn(r)


def _deg_body(dstp_hbm, out_hbm, didx, ones, zb, acc1, sem):
    del sem
    c = lax.axis_index("c")
    s = lax.axis_index("s")
    one16 = jnp.full((16,), 1.0, jnp.float32)
    zero16 = jnp.zeros((16,), jnp.float32)
    for i in range(CHD):
        ones[i, :] = one16
    for i in range(16):
        zb[i, :] = zero16

    def run(r):
        def zbody(i, _):
            pltpu.sync_copy(zb, acc1.at[pl.ds(s * OPT + i * 16, 16)])
            return 0

        lax.fori_loop(0, OPT // 16, zbody, 0)
        pltpu.sync_copy(dstp_hbm.at[pl.ds((r * NS + s) * NCHD, NCHD)], didx)
        plsc.subcore_barrier()

        def body(j, _):
            pltpu.sync_copy(ones, acc1.at[didx.at[j]], add=True)
            return 0

        lax.fori_loop(0, NCHD, body, 0)
        plsc.subcore_barrier()
        pltpu.sync_copy(
            acc1.at[pl.ds(s * OPT, OPT)],
            out_hbm.at[pl.ds(r * NPAD + s * OPT, OPT)],
        )
        plsc.subcore_barrier()

    _for_each_owned_relation(c, run)


def _deg_kernel(dst_pad):
    return pl.kernel(
        _deg_body,
        out_type=jax.ShapeDtypeStruct((R * NPAD, 16), jnp.float32),
        mesh=_mesh(),
        scratch_types=[
            pltpu.VMEM((NCHD, CHD), jnp.int32),
            pltpu.VMEM((CHD, 16), jnp.float32),
            pltpu.VMEM((16, 16), jnp.float32),
            pltpu.VMEM_SHARED((NPAD, 16), jnp.float32),
            pltpu.SemaphoreType.DMA,
        ],
    )(dst_pad)


def _gs_body(
    tbl_hbm, srcp_hbm, dstp_hbm, out_hbm,
    sidx, didx, g0, g1, acc, sga, sgb, ssa, ssb,
):
    c = lax.axis_index("c")
    s = lax.axis_index("s")
    zero16 = jnp.zeros((16,), jnp.float32)

    def run(r):
        def zfill(i, _):
            for j2 in range(F // 16):
                g0[i, pl.ds(j2 * 16, 16)] = zero16
            return 0

        lax.fori_loop(0, CH, zfill, 0)

        def zbody(i, _):
            pltpu.sync_copy(g0, acc.at[pl.ds(s * OPT + i * CH, CH)])
            return 0

        lax.fori_loop(0, OPT // CH, zbody, 0)
        pltpu.sync_copy(srcp_hbm.at[r * NS + s], sidx)
        pltpu.sync_copy(dstp_hbm.at[r * NS + s], didx)
        plsc.subcore_barrier()

        def body(i, _):
            j0 = 2 * i
            j1 = 2 * i + 1
            d0 = pltpu.async_copy(tbl_hbm.at[sidx.at[j0]], g0, sga)
            d1 = pltpu.async_copy(tbl_hbm.at[sidx.at[j1]], g1, sgb)
            d0.wait()
            sc0 = pltpu.make_async_copy(g0, acc.at[didx.at[j0]], ssa)
            sc0.start(add=True)
            d1.wait()
            sc1 = pltpu.make_async_copy(g1, acc.at[didx.at[j1]], ssb)
            sc1.start(add=True)
            sc0.wait()
            sc1.wait()
            return 0

        lax.fori_loop(0, NCH // 2, body, 0)
        plsc.subcore_barrier()
        pltpu.sync_copy(
            acc.at[pl.ds(s * OPT, OPT)],
            out_hbm.at[pl.ds(r * NPAD + s * OPT, OPT)],
        )
        plsc.subcore_barrier()

    _for_each_owned_relation(c, run)


def _gs_kernel(tbl, src_pad, dst_pad):
    return pl.kernel(
        _gs_body,
        out_type=jax.ShapeDtypeStruct((R * NPAD, F), jnp.float32),
        mesh=_mesh(),
        scratch_types=[
            pltpu.VMEM((NCH, CH), jnp.int32),
            pltpu.VMEM((NCH, CH), jnp.int32),
            pltpu.VMEM((CH, F), jnp.float32),
            pltpu.VMEM((CH, F), jnp.float32),
            pltpu.VMEM_SHARED((NPAD, F), jnp.float32),
            pltpu.SemaphoreType.DMA,
            pltpu.SemaphoreType.DMA,
            pltpu.SemaphoreType.DMA,
            pltpu.SemaphoreType.DMA,
        ],
    )(tbl, src_pad, dst_pad)


def _tc1_body(x_ref, wm_ref, bm_ref, wg_ref, degc_ref, h_ref, dinv_ref, tmp_ref):
    h = jnp.maximum(
        jnp.dot(x_ref[...], wm_ref[...], preferred_element_type=jnp.float32)
        + bm_ref[...],
        0.0,
    )
    h_ref[...] = h
    deg = degc_ref[...][:, :, 0] + 1.0
    dinv = lax.rsqrt(deg)
    dinv_ref[...] = dinv
    for r in range(R):
        tmp_ref[r] = (
            jnp.dot(h, wg_ref[r], preferred_element_type=jnp.float32)
            * dinv[:, r : r + 1]
        )


def _tc1(x, w_mlp, b_mlp, wg0, degc):
    return pl.pallas_call(
        _tc1_body,
        grid=(GRID,),
        in_specs=[
            pl.BlockSpec((BLK, F), lambda i: (i, 0)),
            pl.BlockSpec((F, F), lambda i: (0, 0)),
            pl.BlockSpec((1, F), lambda i: (0, 0)),
            pl.BlockSpec((R, F, F), lambda i: (0, 0, 0)),
            pl.BlockSpec((BLK, R, 16), lambda i: (i, 0, 0)),
        ],
        out_specs=[
            pl.BlockSpec((BLK, F), lambda i: (i, 0)),
            pl.BlockSpec((BLK, R), lambda i: (i, 0)),
            pl.BlockSpec((R, BLK, F), lambda i: (0, i, 0)),
        ],
        out_shape=[
            jax.ShapeDtypeStruct((N, F), jnp.float32),
            jax.ShapeDtypeStruct((N, R), jnp.float32),
            jax.ShapeDtypeStruct((R, N, F), jnp.float32),
        ],
    )(x, w_mlp, b_mlp, wg0, degc)


def _post_body(acc_ref, dinv_ref, bg_ref, h_ref, sums_ref):
    i = pl.program_id(0)
    acc = acc_ref[...]
    dinv = dinv_ref[...]
    tot = jnp.zeros((BLK, F), jnp.float32)
    for r in range(R):
        tot = tot + acc[r] * dinv[:, r : r + 1] + bg_ref[r]
    h = tot * (1.0 / R)
    h_ref[...] = h
    stats = jnp.concatenate(
        [
            jnp.sum(h, axis=0, keepdims=True),
            jnp.sum(h * h, axis=0, keepdims=True),
        ],
        axis=0,
    )

    @pl.when(i == 0)
    def _():
        sums_ref[...] = stats

    @pl.when(i > 0)
    def _():
        sums_ref[...] += stats


def _post(acc, dinv, bg):
    return pl.pallas_call(
        _post_body,
        grid=(GRID,),
        in_specs=[
            pl.BlockSpec((R, BLK, F), lambda i: (0, i, 0)),
            pl.BlockSpec((BLK, R), lambda i: (i, 0)),
            pl.BlockSpec((R, 1, F), lambda i: (0, 0, 0)),
        ],
        out_specs=[
            pl.BlockSpec((BLK, F), lambda i: (i, 0)),
            pl.BlockSpec((2, F), lambda i: (0, 0)),
        ],
        out_shape=[
            jax.ShapeDtypeStruct((N, F), jnp.float32),
            jax.ShapeDtypeStruct((2, F), jnp.float32),
        ],
    )(acc, dinv, bg)


def _bn_next_body(hpre_ref, sums_ref, g_ref, b_ref, wg_ref, dinv_ref, tmp_ref):
    mu = sums_ref[0:1, :] * (1.0 / N)
    ex2 = sums_ref[1:2, :] * (1.0 / N)
    var = ex2 - mu * mu
    scale = g_ref[...] * lax.rsqrt(var + 1e-5)
    h = jnp.maximum((hpre_ref[...] - mu) * scale + b_ref[...], 0.0)
    dinv = dinv_ref[...]
    for r in range(R):
        tmp_ref[r] = (
            jnp.dot(h, wg_ref[r], preferred_element_type=jnp.float32)
            * dinv[:, r : r + 1]
        )


def _bn_next(hpre, sums, gamma, beta, wg1, dinv):
    return pl.pallas_call(
        _bn_next_body,
        grid=(GRID,),
        in_specs=[
            pl.BlockSpec((BLK, F), lambda i: (i, 0)),
            pl.BlockSpec((2, F), lambda i: (0, 0)),
            pl.BlockSpec((1, F), lambda i: (0, 0)),
            pl.BlockSpec((1, F), lambda i: (0, 0)),
            pl.BlockSpec((R, F, F), lambda i: (0, 0, 0)),
            pl.BlockSpec((BLK, R), lambda i: (i, 0)),
        ],
        out_specs=[pl.BlockSpec((R, BLK, F), lambda i: (0, i, 0))],
        out_shape=[jax.ShapeDtypeStruct((R, N, F), jnp.float32)],
    )(hpre, sums, gamma, beta, wg1, dinv)[0]


def _final_body(hpre_ref, sums_ref, g_ref, b_ref, res_ref, wl_ref, bl_ref, out_ref):
    mu = sums_ref[0:1, :] * (1.0 / N)
    ex2 = sums_ref[1:2, :] * (1.0 / N)
    var = ex2 - mu * mu
    scale = g_ref[...] * lax.rsqrt(var + 1e-5)
    h = jnp.maximum((hpre_ref[...] - mu) * scale + b_ref[...], 0.0)
    h = h + res_ref[...]
    out_ref[...] = (
        jnp.dot(h, wl_ref[...], preferred_element_type=jnp.float32) + bl_ref[...]
    )


def _final(hpre, sums, gamma, beta, res, w_last, b_last):
    return pl.pallas_call(
        _final_body,
        grid=(GRID,),
        in_specs=[
            pl.BlockSpec((BLK, F), lambda i: (i, 0)),
            pl.BlockSpec((2, F), lambda i: (0, 0)),
            pl.BlockSpec((1, F), lambda i: (0, 0)),
            pl.BlockSpec((1, F), lambda i: (0, 0)),
            pl.BlockSpec((BLK, F), lambda i: (i, 0)),
            pl.BlockSpec((F, OUT), lambda i: (0, 0)),
            pl.BlockSpec((1, OUT), lambda i: (0, 0)),
        ],
        out_specs=[pl.BlockSpec((BLK, OUT), lambda i: (i, 0))],
        out_shape=[jax.ShapeDtypeStruct((N, OUT), jnp.float32)],
    )(hpre, sums, gamma, beta, res, w_last, b_last)[0]


def kernel(x, edge_index, W_mlp, b_mlp, W_gcn, b_gcn, bn_gamma, bn_beta, W_last, b_last):
    ei = edge_index.astype(jnp.int32)
    src = ei[0].reshape(R, NS, EPT)
    dst = ei[1].reshape(R, NS, EPT)

    padd = ((0, 0), (0, 0), (0, NPTD - EPT))
    dst_pad_deg = jnp.pad(dst, padd, constant_values=DUMMY).reshape(
        R * NS * NCHD, CHD
    )

    selfe = jnp.broadcast_to(
        jnp.arange(N, dtype=jnp.int32).reshape(1, NS, SLT), (R, NS, SLT)
    )
    src_full = jnp.concatenate([src, selfe], axis=2)
    dst_full = jnp.concatenate([dst, selfe], axis=2)
    src_full = src_full + (jnp.arange(R, dtype=jnp.int32) * N)[:, None, None]
    pads = ((0, 0), (0, 0), (0, NPT - EPT - SLT))
    src_pad = jnp.pad(src_full, pads, constant_values=0).reshape(R * NS, NCH, CH)
    dst_pad = jnp.pad(dst_full, pads, constant_values=DUMMY).reshape(
        R * NS, NCH, CH
    )

    degc = _deg_kernel(dst_pad_deg).reshape(R, NPAD, 16)[:, :N].transpose(1, 0, 2)

    b_mlp2 = b_mlp.reshape(1, F)
    bg = b_gcn.reshape(2, R, 1, F)
    gamma = bn_gamma.reshape(2, 1, F)
    beta = bn_beta.reshape(2, 1, F)
    h0, dinv, tbl1 = _tc1(x, W_mlp, b_mlp2, W_gcn[0], degc)

    acc = _gs_kernel(tbl1.reshape(R * N, F), src_pad, dst_pad)
    acc = acc.reshape(R, NPAD, F)[:, :N]
    h1pre, sums1 = _post(acc, dinv, bg[0])
    tbl2 = _bn_next(h1pre, sums1, gamma[0], beta[0], W_gcn[1], dinv)

    acc2 = _gs_kernel(tbl2.reshape(R * N, F), src_pad, dst_pad)
    acc2 = acc2.reshape(R, NPAD, F)[:, :N]
    h2pre, sums2 = _post(acc2, dinv, bg[1])
    return _final(
        h2pre, sums2, gamma[1], beta[1], h0, W_last, b_last.reshape(1, OUT)
    )
